# baseline (speedup 1.0000x reference)
import functools

import jax
import jax.numpy as jnp
from jax import lax
from jax.experimental import pallas as pl
from jax.experimental.pallas import tpu as pltpu
from jax.experimental.pallas import tpu_sc as plsc

BATCH = 16384
EMBED = 64
NC = 2
NS = 16
NW = NC * NS
W = 4096
SU = 512
N_SONG = 1000000
N_USER = 100000
TS_SONG = 999936
TS_USER = 99840
TAIL_SONG = N_SONG - TS_SONG
TAIL_USER = N_USER - TS_USER
NSLAB_SONG = 245
NSLAB_USER = 25
TMAX_SONG = 8
TMAX_USER = 1
CH = 64


def _scan_gather(wid, lanes, ids_hbm, tbl_t, out_hbm, nslabs, tmax, ts,
                 idsbuf, mypos, myid, hitpos, hitcol, slabbuf,
                 colstage, fragbuf, sem):
    pltpu.sync_copy(ids_hbm, idsbuf)

    def p1(g, cnt):
        vi = idsbuf[pl.ds(g * 16, 16)]
        m = (((vi >> 12) & 31) == wid) & (vi < ts)
        pf = plsc.cumsum(jnp.where(m, 1, 0))
        dst = cnt + pf - 1
        plsc.store_scatter(mypos, [dst], g * 16 + lanes, mask=m)
        plsc.store_scatter(myid, [dst], vi, mask=m)
        return cnt + pf[15]

    mycnt = lax.fori_loop(0, BATCH // 16, p1, jnp.int32(0))
    myg = (mycnt + 15) >> 4

    def slab_body(t, _):
        s = wid + 32 * t

        @pl.when(s < nslabs)
        def _():
            off = s * W

            def p2(g, cnt):
                vi = myid[pl.ds(g * 16, 16)]
                vp = mypos[pl.ds(g * 16, 16)]
                m = ((vi >> 12) == s) & ((g * 16 + lanes) < mycnt)
                pf = plsc.cumsum(jnp.where(m, 1, 0))
                dst = cnt + pf - 1
                plsc.store_scatter(hitcol, [dst], vi - off, mask=m)
                plsc.store_scatter(hitpos, [dst], vp, mask=m)
                return cnt + pf[15]

            cnt = lax.fori_loop(0, myg, p2, jnp.int32(0))

            def chunk(ch, _):
                bh = ch * CH
                k = jnp.minimum(cnt - bh, CH)
                ng = (k + 15) >> 4
                for b in range(8):
                    for u in range(8):
                        @pl.when(off + (u + 1) * SU <= ts)
                        def _():
                            pltpu.sync_copy(
                                tbl_t.at[pl.ds(8 * b, 8),
                                         pl.ds(off + u * SU, SU)],
                                slabbuf.at[:, pl.ds(u * SU, SU)])

                    def gb(hg, _):
                        cols = hitcol[pl.ds(bh + hg * 16, 16)]
                        msk = (hg * 16 + lanes) < k
                        for r in range(8):
                            vals = plsc.load_gather(
                                slabbuf, [lanes * 0 + r, cols], mask=msk)
                            plsc.store_scatter(
                                colstage, [lanes * 0 + 8 * b + r,
                                           hg * 16 + lanes], vals, mask=msk)
                        return 0

                    lax.fori_loop(0, ng, gb, 0)

                def wb(hg, _):
                    pos16 = hitpos[pl.ds(bh + hg * 16, 16)]
                    for jj in range(16):
                        hidx = hg * 16 + jj

                        @pl.when(hidx < k)
                        def _():
                            for c in range(EMBED // 16):
                                v = plsc.load_gather(
                                    colstage, [c * 16 + lanes,
                                               lanes * 0 + hidx])
                                fragbuf[hidx, pl.ds(c * 16, 16)] = v
                            pltpu.async_copy(
                                fragbuf.at[pl.ds(hidx, 1)],
                                out_hbm.at[pl.ds(pos16[jj], 1)], sem)
                    return 0

                lax.fori_loop(0, ng, wb, 0)

                def db(i, _):
                    pltpu.make_async_copy(out_hbm.at[pl.ds(0, 1)],
                                          fragbuf.at[pl.ds(0, 1)],
                                          sem).wait()
                    return 0

                lax.fori_loop(0, k, db, 0)
                return 0

            lax.fori_loop(0, (cnt + CH - 1) >> 6, chunk, 0)

        return 0

    lax.fori_loop(0, tmax, slab_body, 0)


def _gather_body(users_hbm, songs_hbm, ut_hbm, st_hbm, u_out, s_out,
                 idsbuf, mypos, myid, hitpos, hitcol, slabbuf, colstage,
                 fragbuf, sem):
    wid = lax.axis_index("s") * NC + lax.axis_index("c")
    lanes = lax.broadcasted_iota(jnp.int32, (16,), 0)
    _scan_gather(wid, lanes, users_hbm, ut_hbm, u_out, NSLAB_USER, TMAX_USER,
                 TS_USER, idsbuf, mypos, myid, hitpos, hitcol,
                 slabbuf, colstage, fragbuf, sem)
    _scan_gather(wid, lanes, songs_hbm, st_hbm, s_out, NSLAB_SONG, TMAX_SONG,
                 TS_SONG, idsbuf, mypos, myid, hitpos, hitcol,
                 slabbuf, colstage, fragbuf, sem)


def _sc_gather(users, songs, ut_t, st_t):
    mesh = plsc.VectorSubcoreMesh(core_axis_name="c", subcore_axis_name="s")
    f = pl.kernel(
        _gather_body,
        mesh=mesh,
        compiler_params=pltpu.CompilerParams(needs_layout_passes=False),
        out_type=(
            jax.ShapeDtypeStruct((BATCH, EMBED), jnp.float32),
            jax.ShapeDtypeStruct((BATCH, EMBED), jnp.float32),
        ),
        scratch_types=[
            pltpu.VMEM((BATCH,), jnp.int32),
            pltpu.VMEM((BATCH,), jnp.int32),
            pltpu.VMEM((BATCH,), jnp.int32),
            pltpu.VMEM((BATCH,), jnp.int32),
            pltpu.VMEM((BATCH,), jnp.int32),
            pltpu.VMEM((8, W), jnp.float32),
            pltpu.VMEM((EMBED, CH), jnp.float32),
            pltpu.VMEM((CH, EMBED), jnp.float32),
            pltpu.SemaphoreType.DMA,
        ],
    )
    return f(users, songs, ut_t, st_t)


def _mlp_body(u_ref, s_ref, iu_ref, is_ref, tu_ref, tsg_ref,
              a_ref, b_ref, b1_ref, w2_ref, b2_ref, o_ref):
    du = iu_ref[...] - TS_USER
    dsg = is_ref[...] - TS_SONG
    iota_u = lax.broadcasted_iota(jnp.int32, (1, TAIL_USER), 1)
    iota_s = lax.broadcasted_iota(jnp.int32, (1, TAIL_SONG), 1)
    oh_u = (du == iota_u).astype(jnp.float32)
    oh_s = (dsg == iota_s).astype(jnp.float32)
    dn = (((1,), (0,)), ((), ()))
    fix_u = lax.dot_general(oh_u, tu_ref[...], dn,
                            preferred_element_type=jnp.float32)
    fix_s = lax.dot_general(oh_s, tsg_ref[...], dn,
                            preferred_element_type=jnp.float32)
    u = jnp.where(du >= 0, fix_u, u_ref[...])
    s = jnp.where(dsg >= 0, fix_s, s_ref[...])
    h = lax.dot_general(u, a_ref[...], dn,
                        preferred_element_type=jnp.float32)
    h = h + lax.dot_general(s, b_ref[...], dn,
                            preferred_element_type=jnp.float32)
    h = jnp.maximum(h + b1_ref[...], 0.0)
    o_ref[...] = jnp.sum(h * w2_ref[...], axis=1, keepdims=True) + b2_ref[...]


def _tc_mlp(u_rows, s_rows, iu, isg, tail_u, tail_s, w1a, w1b, b1, w2r, b2):
    bs = 2048
    grid = (BATCH // bs,)
    return pl.pallas_call(
        _mlp_body,
        grid=grid,
        in_specs=[
            pl.BlockSpec((bs, EMBED), lambda i: (i, 0)),
            pl.BlockSpec((bs, EMBED), lambda i: (i, 0)),
            pl.BlockSpec((bs, 1), lambda i: (i, 0)),
            pl.BlockSpec((bs, 1), lambda i: (i, 0)),
            pl.BlockSpec((TAIL_USER, EMBED), lambda i: (0, 0)),
            pl.BlockSpec((TAIL_SONG, EMBED), lambda i: (0, 0)),
            pl.BlockSpec((EMBED, EMBED), lambda i: (0, 0)),
            pl.BlockSpec((EMBED, EMBED), lambda i: (0, 0)),
            pl.BlockSpec((1, EMBED), lambda i: (0, 0)),
            pl.BlockSpec((1, EMBED), lambda i: (0, 0)),
            pl.BlockSpec((1, 1), lambda i: (0, 0)),
        ],
        out_specs=pl.BlockSpec((bs, 1), lambda i: (i, 0)),
        out_shape=jax.ShapeDtypeStruct((BATCH, 1), jnp.float32),
    )(u_rows, s_rows, iu, isg, tail_u, tail_s, w1a, w1b, b1, w2r, b2)


def kernel(users, songs, user_table, song_table, W1, b1, W2, b2):
    users = users.astype(jnp.int32)
    songs = songs.astype(jnp.int32)
    u_rows, s_rows = _sc_gather(users, songs, user_table.T, song_table.T)
    return _tc_mlp(u_rows, s_rows,
                   users.reshape(BATCH, 1), songs.reshape(BATCH, 1),
                   user_table[TS_USER:], song_table[TS_SONG:],
                   W1[:EMBED], W1[EMBED:],
                   b1.reshape(1, EMBED),
                   W2.reshape(1, EMBED),
                   b2.reshape(1, 1))


# speedup vs baseline: 3.5750x; 3.5750x over previous
import functools

import jax
import jax.numpy as jnp
from jax import lax
from jax.experimental import pallas as pl
from jax.experimental.pallas import tpu as pltpu
from jax.experimental.pallas import tpu_sc as plsc

BATCH = 16384
EMBED = 64
NC = 2
NS = 16
NW = NC * NS
W = 4096
SU = 512
N_SONG = 1000000
N_USER = 100000
TS_SONG = 999936
TS_USER = 99840
TAIL_SONG = N_SONG - TS_SONG
TAIL_USER = N_USER - TS_USER
NSLAB_SONG = 245
NSLAB_USER = 25
TMAX_SONG = 8
TMAX_USER = 1
CH = 128
RING = 4


def _scan_gather(wid, lanes, ids_hbm, tbl_t, out_hbm, nslabs, tmax, ts,
                 idsbuf, mypos, myid, hitpos, hitcol, slabbuf,
                 colstage, fragbuf, sem, sem2):
    pltpu.sync_copy(ids_hbm, idsbuf)

    def p1(g, cnt):
        vi = idsbuf[pl.ds(g * 16, 16)]
        m = (((vi >> 12) & 31) == wid) & (vi < ts)
        pf = plsc.cumsum(jnp.where(m, 1, 0))
        dst = cnt + pf - 1
        plsc.store_scatter(mypos, [dst], g * 16 + lanes, mask=m)
        plsc.store_scatter(myid, [dst], vi, mask=m)
        return cnt + pf[15]

    mycnt = lax.fori_loop(0, BATCH // 16, p1, jnp.int32(0))
    myg = (mycnt + 15) >> 4

    def slab_body(t, _):
        s = wid + 32 * t

        @pl.when(s < nslabs)
        def _():
            off = s * W

            def p2(g, cnt):
                vi = myid[pl.ds(g * 16, 16)]
                vp = mypos[pl.ds(g * 16, 16)]
                m = ((vi >> 12) == s) & ((g * 16 + lanes) < mycnt)
                pf = plsc.cumsum(jnp.where(m, 1, 0))
                dst = cnt + pf - 1
                plsc.store_scatter(hitcol, [dst], vi - off, mask=m)
                plsc.store_scatter(hitpos, [dst], vp, mask=m)
                return cnt + pf[15]

            cnt = lax.fori_loop(0, myg, p2, jnp.int32(0))

            def chunk(ch, _):
                bh = ch * CH
                k = jnp.minimum(cnt - bh, CH)
                ng = (k + 15) >> 4

                def unit_slices(bu):
                    b = bu >> 3
                    u = bu & 7
                    src = (pl.ds(8 * b, 8),
                           pl.ds(jnp.minimum(off + u * SU, ts - SU), SU))
                    return src, bu & (RING - 1)

                def issue(bu):
                    src, slot = unit_slices(bu)
                    pltpu.async_copy(tbl_t.at[src], slabbuf.at[slot], sem2)

                def wait_unit(bu):
                    src, slot = unit_slices(bu)
                    pltpu.make_async_copy(tbl_t.at[src], slabbuf.at[slot],
                                          sem2).wait()

                for i in range(RING - 1):
                    issue(jnp.int32(i))

                def ub(bu, _):
                    @pl.when(bu + RING - 1 < 64)
                    def _():
                        issue(bu + RING - 1)

                    wait_unit(bu)
                    b = bu >> 3
                    u = bu & 7
                    slot = bu & (RING - 1)

                    def gb(hg, _):
                        cols = hitcol[pl.ds(bh + hg * 16, 16)]
                        msk = (((hg * 16 + lanes) < k)
                               & ((cols >> 9) == u))
                        vloc = cols & (SU - 1)
                        for r in range(8):
                            vals = plsc.load_gather(
                                slabbuf,
                                [lanes * 0 + slot, lanes * 0 + r, vloc],
                                mask=msk)
                            plsc.store_scatter(
                                colstage, [lanes * 0 + 8 * b + r,
                                           hg * 16 + lanes], vals,
                                mask=msk)
                        return 0

                    lax.fori_loop(0, ng, gb, 0)
                    return 0

                lax.fori_loop(0, 64, ub, 0)

                def wb(hg, _):
                    pos16 = hitpos[pl.ds(bh + hg * 16, 16)]
                    for jj in range(16):
                        hidx = hg * 16 + jj

                        @pl.when(hidx < k)
                        def _():
                            for c in range(EMBED // 16):
                                v = plsc.load_gather(
                                    colstage, [c * 16 + lanes,
                                               lanes * 0 + hidx])
                                fragbuf[hidx, pl.ds(c * 16, 16)] = v
                            pltpu.async_copy(
                                fragbuf.at[pl.ds(hidx, 1)],
                                out_hbm.at[pl.ds(pos16[jj], 1)], sem)
                    return 0

                lax.fori_loop(0, ng, wb, 0)

                def db(i, _):
                    pltpu.make_async_copy(out_hbm.at[pl.ds(0, 1)],
                                          fragbuf.at[pl.ds(0, 1)],
                                          sem).wait()
                    return 0

                lax.fori_loop(0, k, db, 0)
                return 0

            lax.fori_loop(0, (cnt + CH - 1) >> 7, chunk, 0)

        return 0

    lax.fori_loop(0, tmax, slab_body, 0)


def _gather_body(users_hbm, songs_hbm, ut_hbm, st_hbm, u_out, s_out,
                 idsbuf, mypos, myid, hitpos, hitcol, slabbuf, colstage,
                 fragbuf, sem, sem2):
    wid = lax.axis_index("s") * NC + lax.axis_index("c")
    lanes = lax.broadcasted_iota(jnp.int32, (16,), 0)
    _scan_gather(wid, lanes, users_hbm, ut_hbm, u_out, NSLAB_USER, TMAX_USER,
                 TS_USER, idsbuf, mypos, myid, hitpos, hitcol,
                 slabbuf, colstage, fragbuf, sem, sem2)
    _scan_gather(wid, lanes, songs_hbm, st_hbm, s_out, NSLAB_SONG, TMAX_SONG,
                 TS_SONG, idsbuf, mypos, myid, hitpos, hitcol,
                 slabbuf, colstage, fragbuf, sem, sem2)


def _sc_gather(users, songs, ut_t, st_t):
    mesh = plsc.VectorSubcoreMesh(core_axis_name="c", subcore_axis_name="s")
    f = pl.kernel(
        _gather_body,
        mesh=mesh,
        compiler_params=pltpu.CompilerParams(needs_layout_passes=False),
        out_type=(
            jax.ShapeDtypeStruct((BATCH, EMBED), jnp.float32),
            jax.ShapeDtypeStruct((BATCH, EMBED), jnp.float32),
        ),
        scratch_types=[
            pltpu.VMEM((BATCH,), jnp.int32),
            pltpu.VMEM((BATCH,), jnp.int32),
            pltpu.VMEM((BATCH,), jnp.int32),
            pltpu.VMEM((BATCH,), jnp.int32),
            pltpu.VMEM((BATCH,), jnp.int32),
            pltpu.VMEM((RING, 8, SU), jnp.float32),
            pltpu.VMEM((EMBED, CH), jnp.float32),
            pltpu.VMEM((CH, EMBED), jnp.float32),
            pltpu.SemaphoreType.DMA,
            pltpu.SemaphoreType.DMA,
        ],
    )
    return f(users, songs, ut_t, st_t)


def _mlp_body(u_ref, s_ref, iu_ref, is_ref, tu_ref, tsg_ref,
              a_ref, b_ref, b1_ref, w2_ref, b2_ref, o_ref):
    du = iu_ref[...] - TS_USER
    dsg = is_ref[...] - TS_SONG
    iota_u = lax.broadcasted_iota(jnp.int32, (1, TAIL_USER), 1)
    iota_s = lax.broadcasted_iota(jnp.int32, (1, TAIL_SONG), 1)
    oh_u = (du == iota_u).astype(jnp.float32)
    oh_s = (dsg == iota_s).astype(jnp.float32)
    dn = (((1,), (0,)), ((), ()))
    fix_u = lax.dot_general(oh_u, tu_ref[...], dn,
                            preferred_element_type=jnp.float32)
    fix_s = lax.dot_general(oh_s, tsg_ref[...], dn,
                            preferred_element_type=jnp.float32)
    u = jnp.where(du >= 0, fix_u, u_ref[...])
    s = jnp.where(dsg >= 0, fix_s, s_ref[...])
    h = lax.dot_general(u, a_ref[...], dn,
                        preferred_element_type=jnp.float32)
    h = h + lax.dot_general(s, b_ref[...], dn,
                            preferred_element_type=jnp.float32)
    h = jnp.maximum(h + b1_ref[...], 0.0)
    o_ref[...] = jnp.sum(h * w2_ref[...], axis=1, keepdims=True) + b2_ref[...]


def _tc_mlp(u_rows, s_rows, iu, isg, tail_u, tail_s, w1a, w1b, b1, w2r, b2):
    bs = 2048
    grid = (BATCH // bs,)
    return pl.pallas_call(
        _mlp_body,
        grid=grid,
        in_specs=[
            pl.BlockSpec((bs, EMBED), lambda i: (i, 0)),
            pl.BlockSpec((bs, EMBED), lambda i: (i, 0)),
            pl.BlockSpec((bs, 1), lambda i: (i, 0)),
            pl.BlockSpec((bs, 1), lambda i: (i, 0)),
            pl.BlockSpec((TAIL_USER, EMBED), lambda i: (0, 0)),
            pl.BlockSpec((TAIL_SONG, EMBED), lambda i: (0, 0)),
            pl.BlockSpec((EMBED, EMBED), lambda i: (0, 0)),
            pl.BlockSpec((EMBED, EMBED), lambda i: (0, 0)),
            pl.BlockSpec((1, EMBED), lambda i: (0, 0)),
            pl.BlockSpec((1, EMBED), lambda i: (0, 0)),
            pl.BlockSpec((1, 1), lambda i: (0, 0)),
        ],
        out_specs=pl.BlockSpec((bs, 1), lambda i: (i, 0)),
        out_shape=jax.ShapeDtypeStruct((BATCH, 1), jnp.float32),
    )(u_rows, s_rows, iu, isg, tail_u, tail_s, w1a, w1b, b1, w2r, b2)


def kernel(users, songs, user_table, song_table, W1, b1, W2, b2):
    users = users.astype(jnp.int32)
    songs = songs.astype(jnp.int32)
    u_rows, s_rows = _sc_gather(users, songs, user_table.T, song_table.T)
    return _tc_mlp(u_rows, s_rows,
                   users.reshape(BATCH, 1), songs.reshape(BATCH, 1),
                   user_table[TS_USER:], song_table[TS_SONG:],
                   W1[:EMBED], W1[EMBED:],
                   b1.reshape(1, EMBED),
                   W2.reshape(1, EMBED),
                   b2.reshape(1, 1))


# speedup vs baseline: 4.7620x; 1.3320x over previous
import functools

import jax
import jax.numpy as jnp
from jax import lax
from jax.experimental import pallas as pl
from jax.experimental.pallas import tpu as pltpu
from jax.experimental.pallas import tpu_sc as plsc

BATCH = 16384
EMBED = 64
NC = 2
NS = 16
NW = NC * NS
W = 4096
SU = 512
N_SONG = 1000000
N_USER = 100000
TS_SONG = 999936
TS_USER = 99840
TAIL_SONG = N_SONG - TS_SONG
TAIL_USER = N_USER - TS_USER
NSLAB_SONG = 245
NSLAB_USER = 25
TMAX_SONG = 8
TMAX_USER = 1
CH = 256


def _scan_gather(wid, lanes, ids_hbm, tbl_t, out_hbm, nslabs, tmax, ts,
                 mypos, myid, hitpos, hitcol, slabbuf,
                 colstage, fragbuf, sem, sem2):
    pltpu.sync_copy(ids_hbm, mypos)

    def p1(g, cnt):
        vi = mypos[pl.ds(g * 16, 16)]
        m = (((vi >> 12) & 31) == wid) & (vi < ts)
        pf = plsc.cumsum(jnp.where(m, 1, 0))
        dst = cnt + pf - 1
        plsc.store_scatter(myid, [dst], vi, mask=m)
        plsc.store_scatter(mypos, [dst], g * 16 + lanes, mask=m)
        return cnt + pf[15]

    mycnt = lax.fori_loop(0, BATCH // 16, p1, jnp.int32(0))
    myg = (mycnt + 15) >> 4

    def slab_body(t, _):
        s = wid + 32 * t

        @pl.when(s < nslabs)
        def _():
            roff = jnp.minimum(s * W, ts - W)

            def p2a(g, cnt):
                vi = myid[pl.ds(g * 16, 16)]
                m = ((vi >> 12) == s) & ((g * 16 + lanes) < mycnt)
                return cnt + plsc.cumsum(jnp.where(m, 1, 0))[15]

            cnt = lax.fori_loop(0, myg, p2a, jnp.int32(0))

            def chunk(ch, _):
                bh = ch * CH
                k = jnp.minimum(cnt - bh, CH)
                ng = (k + 15) >> 4

                def p2b(g, c2):
                    vi = myid[pl.ds(g * 16, 16)]
                    vp = mypos[pl.ds(g * 16, 16)]
                    m = ((vi >> 12) == s) & ((g * 16 + lanes) < mycnt)
                    pf = plsc.cumsum(jnp.where(m, 1, 0))
                    gi = c2 + pf - 1
                    sel = m & (gi >= bh) & (gi < bh + CH)
                    plsc.store_scatter(hitcol, [gi - bh], vi - roff,
                                       mask=sel)
                    plsc.store_scatter(hitpos, [gi - bh], vp, mask=sel)
                    return c2 + pf[15]

                lax.fori_loop(0, myg, p2b, jnp.int32(0))

                def fb_src(b):
                    return tbl_t.at[pl.ds(4 * b, 4), pl.ds(roff, W)]

                cp = pltpu.async_copy(fb_src(0), slabbuf.at[0], sem2)
                for b in range(16):
                    nxt = None
                    if b < 15:
                        nxt = pltpu.async_copy(fb_src(b + 1),
                                               slabbuf.at[(b + 1) % 2],
                                               sem2)
                    cp.wait()

                    def gb(hg, _):
                        cols = hitcol[pl.ds(hg * 16, 16)]
                        msk = (hg * 16 + lanes) < k
                        for r in range(4):
                            vals = plsc.load_gather(
                                slabbuf,
                                [lanes * 0 + b % 2, lanes * 0 + r, cols],
                                mask=msk)
                            plsc.store_scatter(
                                colstage, [lanes * 0 + 4 * b + r,
                                           hg * 16 + lanes], vals,
                                mask=msk)
                        return 0

                    lax.fori_loop(0, ng, gb, 0)
                    cp = nxt

                def wb(hg, _):
                    pos16 = hitpos[pl.ds(hg * 16, 16)]
                    for jj in range(16):
                        hidx = hg * 16 + jj

                        @pl.when(hidx < k)
                        def _():
                            for c in range(EMBED // 16):
                                v = plsc.load_gather(
                                    colstage, [c * 16 + lanes,
                                               lanes * 0 + hidx])
                                fragbuf[hidx, pl.ds(c * 16, 16)] = v
                            pltpu.async_copy(
                                fragbuf.at[pl.ds(hidx, 1)],
                                out_hbm.at[pl.ds(pos16[jj], 1)], sem)
                    return 0

                lax.fori_loop(0, ng, wb, 0)

                def db(i, _):
                    pltpu.make_async_copy(out_hbm.at[pl.ds(0, 1)],
                                          fragbuf.at[pl.ds(0, 1)],
                                          sem).wait()
                    return 0

                lax.fori_loop(0, k, db, 0)
                return 0

            lax.fori_loop(0, (cnt + CH - 1) >> 8, chunk, 0)

        return 0

    lax.fori_loop(0, tmax, slab_body, 0)


def _gather_body(users_hbm, songs_hbm, ut_hbm, st_hbm, u_out, s_out,
                 mypos, myid, hitpos, hitcol, slabbuf, colstage,
                 fragbuf, sem, sem2):
    wid = lax.axis_index("s") * NC + lax.axis_index("c")
    lanes = lax.broadcasted_iota(jnp.int32, (16,), 0)
    _scan_gather(wid, lanes, users_hbm, ut_hbm, u_out, NSLAB_USER, TMAX_USER,
                 TS_USER, mypos, myid, hitpos, hitcol,
                 slabbuf, colstage, fragbuf, sem, sem2)
    _scan_gather(wid, lanes, songs_hbm, st_hbm, s_out, NSLAB_SONG, TMAX_SONG,
                 TS_SONG, mypos, myid, hitpos, hitcol,
                 slabbuf, colstage, fragbuf, sem, sem2)


def _sc_gather(users, songs, ut_t, st_t):
    mesh = plsc.VectorSubcoreMesh(core_axis_name="c", subcore_axis_name="s")
    f = pl.kernel(
        _gather_body,
        mesh=mesh,
        compiler_params=pltpu.CompilerParams(needs_layout_passes=False),
        out_type=(
            jax.ShapeDtypeStruct((BATCH, EMBED), jnp.float32),
            jax.ShapeDtypeStruct((BATCH, EMBED), jnp.float32),
        ),
        scratch_types=[
            pltpu.VMEM((BATCH,), jnp.int32),
            pltpu.VMEM((BATCH,), jnp.int32),
            pltpu.VMEM((CH,), jnp.int32),
            pltpu.VMEM((CH,), jnp.int32),
            pltpu.VMEM((2, 4, W), jnp.float32),
            pltpu.VMEM((EMBED, CH), jnp.float32),
            pltpu.VMEM((CH, EMBED), jnp.float32),
            pltpu.SemaphoreType.DMA,
            pltpu.SemaphoreType.DMA,
        ],
    )
    return f(users, songs, ut_t, st_t)


def _mlp_body(u_ref, s_ref, iu_ref, is_ref, tu_ref, tsg_ref,
              a_ref, b_ref, b1_ref, w2_ref, b2_ref, o_ref):
    du = iu_ref[...] - TS_USER
    dsg = is_ref[...] - TS_SONG
    iota_u = lax.broadcasted_iota(jnp.int32, (1, TAIL_USER), 1)
    iota_s = lax.broadcasted_iota(jnp.int32, (1, TAIL_SONG), 1)
    oh_u = (du == iota_u).astype(jnp.float32)
    oh_s = (dsg == iota_s).astype(jnp.float32)
    dn = (((1,), (0,)), ((), ()))
    fix_u = lax.dot_general(oh_u, tu_ref[...], dn,
                            preferred_element_type=jnp.float32)
    fix_s = lax.dot_general(oh_s, tsg_ref[...], dn,
                            preferred_element_type=jnp.float32)
    u = jnp.where(du >= 0, fix_u, u_ref[...])
    s = jnp.where(dsg >= 0, fix_s, s_ref[...])
    h = lax.dot_general(u, a_ref[...], dn,
                        preferred_element_type=jnp.float32)
    h = h + lax.dot_general(s, b_ref[...], dn,
                            preferred_element_type=jnp.float32)
    h = jnp.maximum(h + b1_ref[...], 0.0)
    o_ref[...] = jnp.sum(h * w2_ref[...], axis=1, keepdims=True) + b2_ref[...]


def _tc_mlp(u_rows, s_rows, iu, isg, tail_u, tail_s, w1a, w1b, b1, w2r, b2):
    bs = 2048
    grid = (BATCH // bs,)
    return pl.pallas_call(
        _mlp_body,
        grid=grid,
        in_specs=[
            pl.BlockSpec((bs, EMBED), lambda i: (i, 0)),
            pl.BlockSpec((bs, EMBED), lambda i: (i, 0)),
            pl.BlockSpec((bs, 1), lambda i: (i, 0)),
            pl.BlockSpec((bs, 1), lambda i: (i, 0)),
            pl.BlockSpec((TAIL_USER, EMBED), lambda i: (0, 0)),
            pl.BlockSpec((TAIL_SONG, EMBED), lambda i: (0, 0)),
            pl.BlockSpec((EMBED, EMBED), lambda i: (0, 0)),
            pl.BlockSpec((EMBED, EMBED), lambda i: (0, 0)),
            pl.BlockSpec((1, EMBED), lambda i: (0, 0)),
            pl.BlockSpec((1, EMBED), lambda i: (0, 0)),
            pl.BlockSpec((1, 1), lambda i: (0, 0)),
        ],
        out_specs=pl.BlockSpec((bs, 1), lambda i: (i, 0)),
        out_shape=jax.ShapeDtypeStruct((BATCH, 1), jnp.float32),
    )(u_rows, s_rows, iu, isg, tail_u, tail_s, w1a, w1b, b1, w2r, b2)


def kernel(users, songs, user_table, song_table, W1, b1, W2, b2):
    users = users.astype(jnp.int32)
    songs = songs.astype(jnp.int32)
    u_rows, s_rows = _sc_gather(users, songs, user_table.T, song_table.T)
    return _tc_mlp(u_rows, s_rows,
                   users.reshape(BATCH, 1), songs.reshape(BATCH, 1),
                   user_table[TS_USER:], song_table[TS_SONG:],
                   W1[:EMBED], W1[EMBED:],
                   b1.reshape(1, EMBED),
                   W2.reshape(1, EMBED),
                   b2.reshape(1, 1))


# speedup vs baseline: 4.9469x; 1.0388x over previous
import functools

import jax
import jax.numpy as jnp
from jax import lax
from jax.experimental import pallas as pl
from jax.experimental.pallas import tpu as pltpu
from jax.experimental.pallas import tpu_sc as plsc

BATCH = 16384
EMBED = 64
NC = 2
NS = 16
NW = NC * NS
W = 4096
SU = 512
N_SONG = 1000000
N_USER = 100000
TS_SONG = 999936
TS_USER = 99840
TAIL_SONG = N_SONG - TS_SONG
TAIL_USER = N_USER - TS_USER
NSLAB_SONG = 245
NSLAB_USER = 25
TMAX_SONG = 8
TMAX_USER = 1
CH = 256


def _scan_gather(wid, lanes, ids_hbm, tbl_t, out_hbm, nslabs, tmax, ts,
                 mypos, myid, hitpos, hitcol, slabbuf,
                 colstage, fragbuf, sem, sem2):
    pltpu.sync_copy(ids_hbm, mypos)

    def p1(q, cnt):
        vis, pfs, ms = [], [], []
        for i in range(4):
            vi = mypos[pl.ds((q * 4 + i) * 16, 16)]
            m = (((vi >> 12) & 31) == wid) & (vi < ts)
            vis.append(vi)
            ms.append(m)
            pfs.append(plsc.cumsum(jnp.where(m, 1, 0)))
        for i in range(4):
            dst = cnt + pfs[i] - 1
            plsc.store_scatter(myid, [dst], vis[i], mask=ms[i])
            plsc.store_scatter(mypos, [dst], (q * 4 + i) * 16 + lanes,
                               mask=ms[i])
            cnt = cnt + pfs[i][15]
        return cnt

    mycnt = lax.fori_loop(0, BATCH // 64, p1, jnp.int32(0))
    myg = (mycnt + 15) >> 4

    def slab_body(t, _):
        s = wid + 32 * t

        @pl.when(s < nslabs)
        def _():
            roff = jnp.minimum(s * W, ts - W)

            def p2a(g, cnt):
                vi = myid[pl.ds(g * 16, 16)]
                m = ((vi >> 12) == s) & ((g * 16 + lanes) < mycnt)
                return cnt + plsc.cumsum(jnp.where(m, 1, 0))[15]

            cnt = lax.fori_loop(0, myg, p2a, jnp.int32(0))

            def chunk(ch, _):
                bh = ch * CH
                k = jnp.minimum(cnt - bh, CH)
                ng = (k + 15) >> 4

                def p2b(g, c2):
                    vi = myid[pl.ds(g * 16, 16)]
                    vp = mypos[pl.ds(g * 16, 16)]
                    m = ((vi >> 12) == s) & ((g * 16 + lanes) < mycnt)
                    pf = plsc.cumsum(jnp.where(m, 1, 0))
                    gi = c2 + pf - 1
                    sel = m & (gi >= bh) & (gi < bh + CH)
                    plsc.store_scatter(hitcol, [gi - bh], vi - roff,
                                       mask=sel)
                    plsc.store_scatter(hitpos, [gi - bh], vp, mask=sel)
                    return c2 + pf[15]

                lax.fori_loop(0, myg, p2b, jnp.int32(0))

                def fb_src(b):
                    return tbl_t.at[pl.ds(4 * b, 4), pl.ds(roff, W)]

                cp = pltpu.async_copy(fb_src(0), slabbuf.at[0], sem2)
                for b in range(16):
                    nxt = None
                    if b < 15:
                        nxt = pltpu.async_copy(fb_src(b + 1),
                                               slabbuf.at[(b + 1) % 2],
                                               sem2)
                    cp.wait()

                    def gb(hg, _):
                        cols = hitcol[pl.ds(hg * 16, 16)]
                        msk = (hg * 16 + lanes) < k
                        for r in range(4):
                            vals = plsc.load_gather(
                                slabbuf,
                                [lanes * 0 + b % 2, lanes * 0 + r, cols],
                                mask=msk)
                            plsc.store_scatter(
                                colstage, [lanes * 0 + 4 * b + r,
                                           hg * 16 + lanes], vals,
                                mask=msk)
                        return 0

                    lax.fori_loop(0, ng, gb, 0)
                    cp = nxt

                def wb(hg, _):
                    pos16 = hitpos[pl.ds(hg * 16, 16)]
                    for jj in range(16):
                        hidx = hg * 16 + jj

                        @pl.when(hidx < k)
                        def _():
                            for c in range(EMBED // 16):
                                v = plsc.load_gather(
                                    colstage, [c * 16 + lanes,
                                               lanes * 0 + hidx])
                                fragbuf[hidx, pl.ds(c * 16, 16)] = v
                            pltpu.async_copy(
                                fragbuf.at[pl.ds(hidx, 1)],
                                out_hbm.at[pl.ds(pos16[jj], 1)], sem)
                    return 0

                lax.fori_loop(0, ng, wb, 0)

                def db(i, _):
                    pltpu.make_async_copy(out_hbm.at[pl.ds(0, 1)],
                                          fragbuf.at[pl.ds(0, 1)],
                                          sem).wait()
                    return 0

                lax.fori_loop(0, k, db, 0)
                return 0

            lax.fori_loop(0, (cnt + CH - 1) >> 8, chunk, 0)

        return 0

    lax.fori_loop(0, tmax, slab_body, 0)


def _gather_body(users_hbm, songs_hbm, ut_hbm, st_hbm, u_out, s_out,
                 mypos, myid, hitpos, hitcol, slabbuf, colstage,
                 fragbuf, sem, sem2):
    wid = lax.axis_index("s") * NC + lax.axis_index("c")
    lanes = lax.broadcasted_iota(jnp.int32, (16,), 0)
    _scan_gather(wid, lanes, users_hbm, ut_hbm, u_out, NSLAB_USER, TMAX_USER,
                 TS_USER, mypos, myid, hitpos, hitcol,
                 slabbuf, colstage, fragbuf, sem, sem2)
    _scan_gather(wid, lanes, songs_hbm, st_hbm, s_out, NSLAB_SONG, TMAX_SONG,
                 TS_SONG, mypos, myid, hitpos, hitcol,
                 slabbuf, colstage, fragbuf, sem, sem2)


def _sc_gather(users, songs, ut_t, st_t):
    mesh = plsc.VectorSubcoreMesh(core_axis_name="c", subcore_axis_name="s")
    f = pl.kernel(
        _gather_body,
        mesh=mesh,
        compiler_params=pltpu.CompilerParams(needs_layout_passes=False),
        out_type=(
            jax.ShapeDtypeStruct((BATCH, EMBED), jnp.float32),
            jax.ShapeDtypeStruct((BATCH, EMBED), jnp.float32),
        ),
        scratch_types=[
            pltpu.VMEM((BATCH,), jnp.int32),
            pltpu.VMEM((BATCH,), jnp.int32),
            pltpu.VMEM((CH,), jnp.int32),
            pltpu.VMEM((CH,), jnp.int32),
            pltpu.VMEM((2, 4, W), jnp.float32),
            pltpu.VMEM((EMBED, CH), jnp.float32),
            pltpu.VMEM((CH, EMBED), jnp.float32),
            pltpu.SemaphoreType.DMA,
            pltpu.SemaphoreType.DMA,
        ],
    )
    return f(users, songs, ut_t, st_t)


def _mlp_body(u_ref, s_ref, iu_ref, is_ref, tu_ref, tsg_ref,
              a_ref, b_ref, b1_ref, w2_ref, b2_ref, o_ref):
    du = iu_ref[...] - TS_USER
    dsg = is_ref[...] - TS_SONG
    iota_u = lax.broadcasted_iota(jnp.int32, (1, TAIL_USER), 1)
    iota_s = lax.broadcasted_iota(jnp.int32, (1, TAIL_SONG), 1)
    oh_u = (du == iota_u).astype(jnp.float32)
    oh_s = (dsg == iota_s).astype(jnp.float32)
    dn = (((1,), (0,)), ((), ()))
    fix_u = lax.dot_general(oh_u, tu_ref[...], dn,
                            preferred_element_type=jnp.float32)
    fix_s = lax.dot_general(oh_s, tsg_ref[...], dn,
                            preferred_element_type=jnp.float32)
    u = jnp.where(du >= 0, fix_u, u_ref[...])
    s = jnp.where(dsg >= 0, fix_s, s_ref[...])
    h = lax.dot_general(u, a_ref[...], dn,
                        preferred_element_type=jnp.float32)
    h = h + lax.dot_general(s, b_ref[...], dn,
                            preferred_element_type=jnp.float32)
    h = jnp.maximum(h + b1_ref[...], 0.0)
    o_ref[...] = jnp.sum(h * w2_ref[...], axis=1, keepdims=True) + b2_ref[...]


def _tc_mlp(u_rows, s_rows, iu, isg, tail_u, tail_s, w1a, w1b, b1, w2r, b2):
    bs = 2048
    grid = (BATCH // bs,)
    return pl.pallas_call(
        _mlp_body,
        grid=grid,
        in_specs=[
            pl.BlockSpec((bs, EMBED), lambda i: (i, 0)),
            pl.BlockSpec((bs, EMBED), lambda i: (i, 0)),
            pl.BlockSpec((bs, 1), lambda i: (i, 0)),
            pl.BlockSpec((bs, 1), lambda i: (i, 0)),
            pl.BlockSpec((TAIL_USER, EMBED), lambda i: (0, 0)),
            pl.BlockSpec((TAIL_SONG, EMBED), lambda i: (0, 0)),
            pl.BlockSpec((EMBED, EMBED), lambda i: (0, 0)),
            pl.BlockSpec((EMBED, EMBED), lambda i: (0, 0)),
            pl.BlockSpec((1, EMBED), lambda i: (0, 0)),
            pl.BlockSpec((1, EMBED), lambda i: (0, 0)),
            pl.BlockSpec((1, 1), lambda i: (0, 0)),
        ],
        out_specs=pl.BlockSpec((bs, 1), lambda i: (i, 0)),
        out_shape=jax.ShapeDtypeStruct((BATCH, 1), jnp.float32),
    )(u_rows, s_rows, iu, isg, tail_u, tail_s, w1a, w1b, b1, w2r, b2)


def kernel(users, songs, user_table, song_table, W1, b1, W2, b2):
    users = users.astype(jnp.int32)
    songs = songs.astype(jnp.int32)
    u_rows, s_rows = _sc_gather(users, songs, user_table.T, song_table.T)
    return _tc_mlp(u_rows, s_rows,
                   users.reshape(BATCH, 1), songs.reshape(BATCH, 1),
                   user_table[TS_USER:], song_table[TS_SONG:],
                   W1[:EMBED], W1[EMBED:],
                   b1.reshape(1, EMBED),
                   W2.reshape(1, EMBED),
                   b2.reshape(1, 1))


# speedup vs baseline: 4.9841x; 1.0075x over previous
import functools

import jax
import jax.numpy as jnp
from jax import lax
from jax.experimental import pallas as pl
from jax.experimental.pallas import tpu as pltpu
from jax.experimental.pallas import tpu_sc as plsc

BATCH = 16384
EMBED = 64
NC = 2
NS = 16
NW = NC * NS
W = 4096
SU = 512
N_SONG = 1000000
N_USER = 100000
TS_SONG = 999936
TS_USER = 99840
TAIL_SONG = N_SONG - TS_SONG
TAIL_USER = N_USER - TS_USER
NSLAB_SONG = 245
NSLAB_USER = 25
TMAX_SONG = 8
TMAX_USER = 1
CH = 256


def _scan_gather(wid, lanes, ids_hbm, tbl_t, out_hbm, nslabs, tmax, ts,
                 mypos, myid, hitpos, hitcol, slabbuf,
                 colstage, fragbuf, sem, sem2):
    pltpu.sync_copy(ids_hbm, mypos)

    def p1(q, cnt):
        vis, pfs, ms = [], [], []
        for i in range(4):
            vi = mypos[pl.ds((q * 4 + i) * 16, 16)]
            m = (((vi >> 12) & 31) == wid) & (vi < ts)
            vis.append(vi)
            ms.append(m)
            pfs.append(plsc.cumsum(jnp.where(m, 1, 0)))
        for i in range(4):
            dst = cnt + pfs[i] - 1
            plsc.store_scatter(myid, [dst], vis[i], mask=ms[i])
            plsc.store_scatter(mypos, [dst], (q * 4 + i) * 16 + lanes,
                               mask=ms[i])
            cnt = cnt + pfs[i][15]
        return cnt

    mycnt = lax.fori_loop(0, BATCH // 64, p1, jnp.int32(0))
    myg = (mycnt + 15) >> 4

    def slab_body(t, _):
        s = wid + 32 * t

        @pl.when(s < nslabs)
        def _():
            roff = jnp.minimum(s * W, ts - W)

            def p2b_at(bh):
                def p2b(g, c2):
                    vi = myid[pl.ds(g * 16, 16)]
                    vp = mypos[pl.ds(g * 16, 16)]
                    m = ((vi >> 12) == s) & ((g * 16 + lanes) < mycnt)
                    pf = plsc.cumsum(jnp.where(m, 1, 0))
                    gi = c2 + pf - 1
                    sel = m & (gi >= bh) & (gi < bh + CH)
                    plsc.store_scatter(hitcol, [gi - bh], vi - roff,
                                       mask=sel)
                    plsc.store_scatter(hitpos, [gi - bh], vp, mask=sel)
                    return c2 + pf[15]

                return lax.fori_loop(0, myg, p2b, jnp.int32(0))

            cnt = p2b_at(jnp.int32(0))

            def chunk(ch, _):
                bh = ch * CH
                k = jnp.minimum(cnt - bh, CH)
                ng = (k + 15) >> 4

                @pl.when(ch > 0)
                def _():
                    p2b_at(bh)

                def fb_src(b):
                    return tbl_t.at[pl.ds(4 * b, 4), pl.ds(roff, W)]

                cp = pltpu.async_copy(fb_src(0), slabbuf.at[0], sem2)
                for b in range(16):
                    nxt = None
                    if b < 15:
                        nxt = pltpu.async_copy(fb_src(b + 1),
                                               slabbuf.at[(b + 1) % 2],
                                               sem2)
                    cp.wait()

                    def gb(hg, _):
                        cols = hitcol[pl.ds(hg * 16, 16)]
                        msk = (hg * 16 + lanes) < k
                        for r in range(4):
                            vals = plsc.load_gather(
                                slabbuf,
                                [lanes * 0 + b % 2, lanes * 0 + r, cols],
                                mask=msk)
                            plsc.store_scatter(
                                colstage, [lanes * 0 + 4 * b + r,
                                           hg * 16 + lanes], vals,
                                mask=msk)
                        return 0

                    lax.fori_loop(0, ng, gb, 0)
                    cp = nxt

                def wb(hg, _):
                    pos16 = hitpos[pl.ds(hg * 16, 16)]
                    for jj in range(16):
                        hidx = hg * 16 + jj

                        @pl.when(hidx < k)
                        def _():
                            for c in range(EMBED // 16):
                                v = plsc.load_gather(
                                    colstage, [c * 16 + lanes,
                                               lanes * 0 + hidx])
                                fragbuf[hidx, pl.ds(c * 16, 16)] = v
                            pltpu.async_copy(
                                fragbuf.at[pl.ds(hidx, 1)],
                                out_hbm.at[pl.ds(pos16[jj], 1)], sem)
                    return 0

                lax.fori_loop(0, ng, wb, 0)

                def db(i, _):
                    pltpu.make_async_copy(out_hbm.at[pl.ds(0, 1)],
                                          fragbuf.at[pl.ds(0, 1)],
                                          sem).wait()
                    return 0

                lax.fori_loop(0, k, db, 0)
                return 0

            lax.fori_loop(0, (cnt + CH - 1) >> 8, chunk, 0)

        return 0

    lax.fori_loop(0, tmax, slab_body, 0)


def _gather_body(users_hbm, songs_hbm, ut_hbm, st_hbm, u_out, s_out,
                 mypos, myid, hitpos, hitcol, slabbuf, colstage,
                 fragbuf, sem, sem2):
    wid = lax.axis_index("s") * NC + lax.axis_index("c")
    lanes = lax.broadcasted_iota(jnp.int32, (16,), 0)
    _scan_gather(wid, lanes, users_hbm, ut_hbm, u_out, NSLAB_USER, TMAX_USER,
                 TS_USER, mypos, myid, hitpos, hitcol,
                 slabbuf, colstage, fragbuf, sem, sem2)
    _scan_gather(wid, lanes, songs_hbm, st_hbm, s_out, NSLAB_SONG, TMAX_SONG,
                 TS_SONG, mypos, myid, hitpos, hitcol,
                 slabbuf, colstage, fragbuf, sem, sem2)


def _sc_gather(users, songs, ut_t, st_t):
    mesh = plsc.VectorSubcoreMesh(core_axis_name="c", subcore_axis_name="s")
    f = pl.kernel(
        _gather_body,
        mesh=mesh,
        compiler_params=pltpu.CompilerParams(needs_layout_passes=False),
        out_type=(
            jax.ShapeDtypeStruct((BATCH, EMBED), jnp.float32),
            jax.ShapeDtypeStruct((BATCH, EMBED), jnp.float32),
        ),
        scratch_types=[
            pltpu.VMEM((BATCH,), jnp.int32),
            pltpu.VMEM((BATCH,), jnp.int32),
            pltpu.VMEM((CH,), jnp.int32),
            pltpu.VMEM((CH,), jnp.int32),
            pltpu.VMEM((2, 4, W), jnp.float32),
            pltpu.VMEM((EMBED, CH), jnp.float32),
            pltpu.VMEM((CH, EMBED), jnp.float32),
            pltpu.SemaphoreType.DMA,
            pltpu.SemaphoreType.DMA,
        ],
    )
    return f(users, songs, ut_t, st_t)


def _mlp_body(u_ref, s_ref, iu_ref, is_ref, tu_ref, tsg_ref,
              a_ref, b_ref, b1_ref, w2_ref, b2_ref, o_ref):
    du = iu_ref[...] - TS_USER
    dsg = is_ref[...] - TS_SONG
    iota_u = lax.broadcasted_iota(jnp.int32, (1, TAIL_USER), 1)
    iota_s = lax.broadcasted_iota(jnp.int32, (1, TAIL_SONG), 1)
    oh_u = (du == iota_u).astype(jnp.float32)
    oh_s = (dsg == iota_s).astype(jnp.float32)
    dn = (((1,), (0,)), ((), ()))
    fix_u = lax.dot_general(oh_u, tu_ref[...], dn,
                            preferred_element_type=jnp.float32)
    fix_s = lax.dot_general(oh_s, tsg_ref[...], dn,
                            preferred_element_type=jnp.float32)
    u = jnp.where(du >= 0, fix_u, u_ref[...])
    s = jnp.where(dsg >= 0, fix_s, s_ref[...])
    h = lax.dot_general(u, a_ref[...], dn,
                        preferred_element_type=jnp.float32)
    h = h + lax.dot_general(s, b_ref[...], dn,
                            preferred_element_type=jnp.float32)
    h = jnp.maximum(h + b1_ref[...], 0.0)
    o_ref[...] = jnp.sum(h * w2_ref[...], axis=1, keepdims=True) + b2_ref[...]


def _tc_mlp(u_rows, s_rows, iu, isg, tail_u, tail_s, w1a, w1b, b1, w2r, b2):
    bs = 2048
    grid = (BATCH // bs,)
    return pl.pallas_call(
        _mlp_body,
        grid=grid,
        in_specs=[
            pl.BlockSpec((bs, EMBED), lambda i: (i, 0)),
            pl.BlockSpec((bs, EMBED), lambda i: (i, 0)),
            pl.BlockSpec((bs, 1), lambda i: (i, 0)),
            pl.BlockSpec((bs, 1), lambda i: (i, 0)),
            pl.BlockSpec((TAIL_USER, EMBED), lambda i: (0, 0)),
            pl.BlockSpec((TAIL_SONG, EMBED), lambda i: (0, 0)),
            pl.BlockSpec((EMBED, EMBED), lambda i: (0, 0)),
            pl.BlockSpec((EMBED, EMBED), lambda i: (0, 0)),
            pl.BlockSpec((1, EMBED), lambda i: (0, 0)),
            pl.BlockSpec((1, EMBED), lambda i: (0, 0)),
            pl.BlockSpec((1, 1), lambda i: (0, 0)),
        ],
        out_specs=pl.BlockSpec((bs, 1), lambda i: (i, 0)),
        out_shape=jax.ShapeDtypeStruct((BATCH, 1), jnp.float32),
    )(u_rows, s_rows, iu, isg, tail_u, tail_s, w1a, w1b, b1, w2r, b2)


def kernel(users, songs, user_table, song_table, W1, b1, W2, b2):
    users = users.astype(jnp.int32)
    songs = songs.astype(jnp.int32)
    u_rows, s_rows = _sc_gather(users, songs, user_table.T, song_table.T)
    return _tc_mlp(u_rows, s_rows,
                   users.reshape(BATCH, 1), songs.reshape(BATCH, 1),
                   user_table[TS_USER:], song_table[TS_SONG:],
                   W1[:EMBED], W1[EMBED:],
                   b1.reshape(1, EMBED),
                   W2.reshape(1, EMBED),
                   b2.reshape(1, 1))
